# baseline (speedup 1.0000x reference)
import jax
import jax.numpy as jnp
from jax import lax
from jax.experimental import pallas as pl
from jax.experimental.pallas import tpu as pltpu
from jax.experimental.pallas import tpu_sc as plsc

NUM_CORES = 2
NUM_SUBCORES = 16
LANES = 16
NW = NUM_CORES * NUM_SUBCORES

F = 26
V = 100000
D = 32
B = 16384

MR = V // 4
VCH = 1408
NCHV = 71
VMAIN = VCH * NCHV

SPT = B // NW
CH = 128
NCH = SPT // CH
XPT = SPT * F


def _conv_body(tabt_hbm, tail_hbm, out_hbm, staged_v, outbuf_v, sem):
    wid = lax.axis_index("s") * NUM_CORES + lax.axis_index("c")
    lane32 = lax.iota(jnp.int32, 16) * D

    @pl.loop(wid, F * NCHV, step=NW)
    def _unit(u):
        f = lax.div(u, NCHV)
        v0 = lax.rem(u, NCHV) * VCH

        @pl.loop(0, 4)
        def _ld(dblk):
            pltpu.async_copy(
                tabt_hbm.at[f, pl.ds(dblk * 8, 8), pl.ds(v0, VCH)],
                staged_v.at[dblk], sem)

        @pl.loop(0, 4)
        def _ldd(dblk):
            pltpu.make_async_copy(
                tabt_hbm.at[0, pl.ds(0, 8), pl.ds(0, VCH)],
                staged_v.at[0], sem).wait()

        @pl.loop(0, D)
        def _d(d):
            pos0 = lane32 + d
            row0 = pos0 >> 7
            col0 = pos0 & 127

            @plsc.parallel_loop(0, VCH // LANES, unroll=8)
            def _vg(vg):
                vals = staged_v[d >> 3, d & 7, pl.ds(vg * LANES, LANES)]
                plsc.store_scatter(outbuf_v, [row0 + vg * 4, col0], vals)

        pltpu.sync_copy(
            outbuf_v,
            out_hbm.at[pl.ds(f * MR + lax.div(v0, 4), VCH // 4)])

    @pl.loop(wid, F, step=NW)
    def _tail(f):
        pltpu.sync_copy(tail_hbm.at[f],
                        out_hbm.at[pl.ds(f * MR + VMAIN // 4, 8)])


def _gather_body(tab_hbm, x_hbm, out_hbm, x_v, idx_v, acc_v, sem):
    wid = lax.axis_index("s") * NUM_CORES + lax.axis_index("c")
    pltpu.sync_copy(x_hbm.at[pl.ds(wid * XPT, XPT)], x_v)

    lane_f = lax.iota(jnp.int32, 16) * F

    def _build_row(j):
        i = lax.div(j, NCH)
        c = lax.rem(j, NCH)
        base = c * CH * F + i
        off = i * V

        @pl.loop(0, CH // LANES)
        def _seg(r):
            xi = lane_f + (base + r * LANES * F)
            vals = plsc.load_gather(x_v, [xi])
            idx_v[j, pl.ds(r * LANES, LANES)] = vals + off

    @pl.loop(0, NCH)
    def _build0(j):
        _build_row(j)

    @pl.loop(0, NCH)
    def _init(c):
        pltpu.async_copy(tab_hbm.at[idx_v.at[c]],
                         acc_v.at[pl.ds(c * CH, CH)], sem)

    @pl.loop(NCH, F * NCH)
    def _build(j):
        _build_row(j)

    @pl.loop(0, NCH)
    def _init_drain(c):
        pltpu.make_async_copy(tab_hbm.at[idx_v.at[0]],
                              acc_v.at[pl.ds(0, CH)], sem).wait()

    @pl.loop(NCH, F * NCH)
    def _fire(j):
        c = lax.rem(j, NCH)
        pltpu.async_copy(tab_hbm.at[idx_v.at[j]],
                         acc_v.at[pl.ds(c * CH, CH)], sem, add=True)

    @pl.loop(NCH, F * NCH)
    def _drain(j):
        pltpu.make_async_copy(tab_hbm.at[idx_v.at[0]],
                              acc_v.at[pl.ds(0, CH)], sem).wait()

    pltpu.sync_copy(acc_v, out_hbm.at[pl.ds(wid * SPT, SPT)])


_SC_MESH = dict(core_axis_name="c", subcore_axis_name="s",
                num_cores=NUM_CORES, num_subcores=NUM_SUBCORES)


@jax.jit
def kernel(x, tables):
    tabt = jnp.swapaxes(tables, 1, 2)
    tail = tables[:, VMAIN:, :].reshape(F, 8, 4 * D)
    x_flat = x.reshape(B * F)

    conv = pl.kernel(
        _conv_body,
        out_type=jax.ShapeDtypeStruct((F * MR, 4 * D), jnp.float32),
        compiler_params=pltpu.CompilerParams(use_tc_tiling_on_sc=True,
                                             needs_layout_passes=False),
        mesh=plsc.VectorSubcoreMesh(**_SC_MESH),
        scratch_types=[
            pltpu.VMEM((4, 8, VCH), jnp.float32),
            pltpu.VMEM((VCH // 4, 4 * D), jnp.float32),
            pltpu.SemaphoreType.DMA,
        ],
    )
    tab_l = conv(tabt, tail).reshape(F * V, D)

    gat = pl.kernel(
        _gather_body,
        out_type=jax.ShapeDtypeStruct((B, D), jnp.float32),
        compiler_params=pltpu.CompilerParams(use_tc_tiling_on_sc=False,
                                             needs_layout_passes=False),
        mesh=plsc.VectorSubcoreMesh(**_SC_MESH),
        scratch_types=[
            pltpu.VMEM((XPT,), jnp.int32),
            pltpu.VMEM((F * NCH, CH), jnp.int32),
            pltpu.VMEM((SPT, D), jnp.float32),
            pltpu.SemaphoreType.DMA,
        ],
    )
    return gat(tab_l, x_flat)


# speedup vs baseline: 1.0913x; 1.0913x over previous
import jax
import jax.numpy as jnp
from jax import lax
from jax.experimental import pallas as pl
from jax.experimental.pallas import tpu as pltpu
from jax.experimental.pallas import tpu_sc as plsc

NUM_CORES = 2
NUM_SUBCORES = 16
LANES = 16
NW = NUM_CORES * NUM_SUBCORES

F = 26
V = 100000
D = 32
B = 16384

MR = V // 4
VCH = 896
NCHV = 111
VEPI = 512
VMAIN = VCH * NCHV
VTAIL = VMAIN + VEPI

NU = F * NCHV
NP = (NU + 2 * NW - 1) // (2 * NW)

SPT = B // NW
CH = 128
NCH = SPT // CH
XPT = SPT * F


def _conv_body(tabt_hbm, tail_hbm, out_hbm,
               staged0, staged1, outbuf0, outbuf1,
               sem_l0, sem_l1, sem_w0, sem_w1):
    wid = lax.axis_index("s") * NUM_CORES + lax.axis_index("c")
    lane32 = lax.iota(jnp.int32, 16) * D
    stageds = (staged0, staged1)
    sem_ls = (sem_l0, sem_l1)
    outbufs = (outbuf0, outbuf1)
    sem_ws = (sem_w0, sem_w1)

    def unit(p, parity):
        return jnp.minimum(wid + (2 * p + parity) * NW, NU - 1)

    def fire_loads(u, k):
        f = lax.div(u, NCHV)
        v0 = lax.rem(u, NCHV) * VCH

        @pl.loop(0, 4)
        def _ld(dblk):
            pltpu.async_copy(
                tabt_hbm.at[f, pl.ds(dblk * 8, 8), pl.ds(v0, VCH)],
                stageds[k].at[dblk], sem_ls[k])

    def drain_loads(k):
        @pl.loop(0, 4)
        def _dr(dblk):
            pltpu.make_async_copy(
                tabt_hbm.at[0, pl.ds(0, 8), pl.ds(0, VCH)],
                stageds[k].at[0], sem_ls[k]).wait()

    def out_rows(u):
        f = lax.div(u, NCHV)
        v0 = lax.rem(u, NCHV) * VCH
        return f * MR + lax.div(v0, 4)

    def transpose(k, nvg):
        @pl.loop(0, D)
        def _d(d):
            pos0 = lane32 + d
            row0 = pos0 >> 7
            col0 = pos0 & 127

            @plsc.parallel_loop(0, nvg, unroll=8)
            def _vg(vg):
                vals = stageds[k][d >> 3, d & 7, pl.ds(vg * LANES, LANES)]
                plsc.store_scatter(outbufs[k], [row0 + vg * 4, col0], vals)

    def drain_write(k):
        pltpu.make_async_copy(out_hbm.at[pl.ds(0, VCH // 4)],
                              outbufs[k], sem_ws[k]).wait()

    fire_loads(unit(0, 0), 0)
    fire_loads(unit(0, 1), 1)
    pltpu.async_copy(out_hbm.at[pl.ds(0, VCH // 4)], outbuf0, sem_w0)
    pltpu.async_copy(out_hbm.at[pl.ds(0, VCH // 4)], outbuf1, sem_w1)

    @pl.loop(0, NP)
    def _pipe(p):
        for k in (0, 1):
            u = unit(p, k)
            drain_loads(k)
            drain_write(k)
            transpose(k, VCH // LANES)
            pltpu.async_copy(outbufs[k],
                             out_hbm.at[pl.ds(out_rows(u), VCH // 4)],
                             sem_ws[k])

            @pl.when(p + 1 < NP)
            def _():
                fire_loads(unit(p + 1, k), k)

    drain_write(0)
    drain_write(1)

    f = jnp.minimum(wid, F - 1)

    @pl.loop(0, 4)
    def _eld(dblk):
        pltpu.async_copy(
            tabt_hbm.at[f, pl.ds(dblk * 8, 8), pl.ds(VMAIN, VEPI)],
            staged0.at[dblk, :, pl.ds(0, VEPI)], sem_l0)

    @pl.loop(0, 4)
    def _edr(dblk):
        pltpu.make_async_copy(
            tabt_hbm.at[0, pl.ds(0, 8), pl.ds(VMAIN, VEPI)],
            staged0.at[0, :, pl.ds(0, VEPI)], sem_l0).wait()

    @pl.loop(0, D)
    def _ed(d):
        pos0 = lane32 + d
        row0 = pos0 >> 7
        col0 = pos0 & 127

        @plsc.parallel_loop(0, VEPI // LANES, unroll=8)
        def _evg(vg):
            vals = staged0[d >> 3, d & 7, pl.ds(vg * LANES, LANES)]
            plsc.store_scatter(outbuf0, [row0 + vg * 4, col0], vals)

    pltpu.sync_copy(outbuf0.at[pl.ds(0, VEPI // 4)],
                    out_hbm.at[pl.ds(f * MR + VMAIN // 4, VEPI // 4)])

    @pl.loop(wid, F, step=NW)
    def _tail(ff):
        pltpu.sync_copy(tail_hbm.at[ff],
                        out_hbm.at[pl.ds(ff * MR + VTAIL // 4, 8)])


def _gather_body(tab_hbm, x_hbm, out_hbm, x_v, idx_v, acc_v, sem):
    wid = lax.axis_index("s") * NUM_CORES + lax.axis_index("c")
    pltpu.sync_copy(x_hbm.at[pl.ds(wid * XPT, XPT)], x_v)

    lane_f = lax.iota(jnp.int32, 16) * F

    def _build_row(j):
        i = lax.div(j, NCH)
        c = lax.rem(j, NCH)
        base = c * CH * F + i
        off = i * V

        @pl.loop(0, CH // LANES)
        def _seg(r):
            xi = lane_f + (base + r * LANES * F)
            vals = plsc.load_gather(x_v, [xi])
            idx_v[j, pl.ds(r * LANES, LANES)] = vals + off

    @pl.loop(0, NCH)
    def _build0(j):
        _build_row(j)

    @pl.loop(0, NCH)
    def _init(c):
        pltpu.async_copy(tab_hbm.at[idx_v.at[c]],
                         acc_v.at[pl.ds(c * CH, CH)], sem)

    @pl.loop(NCH, F * NCH)
    def _build(j):
        _build_row(j)

    @pl.loop(0, NCH)
    def _init_drain(c):
        pltpu.make_async_copy(tab_hbm.at[idx_v.at[0]],
                              acc_v.at[pl.ds(0, CH)], sem).wait()

    @pl.loop(NCH, F * NCH)
    def _fire(j):
        c = lax.rem(j, NCH)
        pltpu.async_copy(tab_hbm.at[idx_v.at[j]],
                         acc_v.at[pl.ds(c * CH, CH)], sem, add=True)

    @pl.loop(NCH, F * NCH)
    def _drain(j):
        pltpu.make_async_copy(tab_hbm.at[idx_v.at[0]],
                              acc_v.at[pl.ds(0, CH)], sem).wait()

    pltpu.sync_copy(acc_v, out_hbm.at[pl.ds(wid * SPT, SPT)])


_SC_MESH = dict(core_axis_name="c", subcore_axis_name="s",
                num_cores=NUM_CORES, num_subcores=NUM_SUBCORES)


@jax.jit
def kernel(x, tables):
    tabt = jnp.swapaxes(tables, 1, 2)
    tail = tables[:, VTAIL:, :].reshape(F, 8, 4 * D)
    x_flat = x.reshape(B * F)

    conv = pl.kernel(
        _conv_body,
        out_type=jax.ShapeDtypeStruct((F * MR, 4 * D), jnp.float32),
        compiler_params=pltpu.CompilerParams(use_tc_tiling_on_sc=True,
                                             needs_layout_passes=False),
        mesh=plsc.VectorSubcoreMesh(**_SC_MESH),
        scratch_types=[
            pltpu.VMEM((4, 8, VCH), jnp.float32),
            pltpu.VMEM((4, 8, VCH), jnp.float32),
            pltpu.VMEM((VCH // 4, 4 * D), jnp.float32),
            pltpu.VMEM((VCH // 4, 4 * D), jnp.float32),
            pltpu.SemaphoreType.DMA,
            pltpu.SemaphoreType.DMA,
            pltpu.SemaphoreType.DMA,
            pltpu.SemaphoreType.DMA,
        ],
    )
    tab_l = conv(tabt, tail).reshape(F * V, D)

    gat = pl.kernel(
        _gather_body,
        out_type=jax.ShapeDtypeStruct((B, D), jnp.float32),
        compiler_params=pltpu.CompilerParams(use_tc_tiling_on_sc=False,
                                             needs_layout_passes=False),
        mesh=plsc.VectorSubcoreMesh(**_SC_MESH),
        scratch_types=[
            pltpu.VMEM((XPT,), jnp.int32),
            pltpu.VMEM((F * NCH, CH), jnp.int32),
            pltpu.VMEM((SPT, D), jnp.float32),
            pltpu.SemaphoreType.DMA,
        ],
    )
    return gat(tab_l, x_flat)


# speedup vs baseline: 1.5798x; 1.4476x over previous
import jax
import jax.numpy as jnp
from jax import lax
from jax.experimental import pallas as pl
from jax.experimental.pallas import tpu as pltpu
from jax.experimental.pallas import tpu_sc as plsc

NUM_CORES = 2
NUM_SUBCORES = 16
LANES = 16
NW = NUM_CORES * NUM_SUBCORES

F = 26
V = 100000
D = 32
B = 16384

MR = V // 4
VCH = 896
NCHV = 111
VEPI = 512
VMAIN = VCH * NCHV
VTAIL = VMAIN + VEPI

NU = F * NCHV
NP = (NU + 2 * NW - 1) // (2 * NW)

SPT = B // NW
CH = 128
NCH = SPT // CH
XPT = SPT * F


def _conv_body(tabt_hbm, tail_hbm, out_hbm,
               staged0, staged1, outbuf0, outbuf1,
               sem_l0, sem_l1, sem_w0, sem_w1):
    wid = lax.axis_index("s") * NUM_CORES + lax.axis_index("c")
    lane = lax.iota(jnp.int32, 16)
    stageds = (staged0, staged1)
    sem_ls = (sem_l0, sem_l1)
    outbufs = (outbuf0, outbuf1)
    sem_ws = (sem_w0, sem_w1)

    def unit(p, parity):
        return jnp.minimum(wid + (2 * p + parity) * NW, NU - 1)

    def fire_loads(u, k):
        f = lax.div(u, NCHV)
        v0 = lax.rem(u, NCHV) * VCH

        @pl.loop(0, 4)
        def _ld(dblk):
            pltpu.async_copy(
                tabt_hbm.at[f, pl.ds(dblk * 8, 8), pl.ds(v0, VCH)],
                stageds[k].at[dblk, :, pl.ds(0, VCH)], sem_ls[k])

    def drain_loads(k):
        @pl.loop(0, 4)
        def _dr(dblk):
            pltpu.make_async_copy(
                tabt_hbm.at[0, pl.ds(0, 8), pl.ds(0, VCH)],
                stageds[k].at[0, :, pl.ds(0, VCH)], sem_ls[k]).wait()

    def out_rows(u):
        f = lax.div(u, NCHV)
        v0 = lax.rem(u, NCHV) * VCH
        return f * MR + lax.div(v0, 4)

    blk_lo = lane >> 3
    blk_hi = blk_lo + 2
    dsub = lane & 7

    def transpose(k, nv):
        @plsc.parallel_loop(0, nv, unroll=8)
        def _v(v):
            vb = jnp.zeros((LANES,), jnp.int32) + v
            row = v >> 2
            col = (v & 3) * D
            outbufs[k][row, pl.ds(col, LANES)] = plsc.load_gather(
                stageds[k], [blk_lo, dsub, vb])
            outbufs[k][row, pl.ds(col + LANES, LANES)] = plsc.load_gather(
                stageds[k], [blk_hi, dsub, vb])

    def drain_write(k):
        pltpu.make_async_copy(out_hbm.at[pl.ds(0, VCH // 4)],
                              outbufs[k], sem_ws[k]).wait()

    fire_loads(unit(0, 0), 0)
    fire_loads(unit(0, 1), 1)
    pltpu.async_copy(out_hbm.at[pl.ds(0, VCH // 4)], outbuf0, sem_w0)
    pltpu.async_copy(out_hbm.at[pl.ds(0, VCH // 4)], outbuf1, sem_w1)

    @pl.loop(0, NP)
    def _pipe(p):
        for k in (0, 1):
            u = unit(p, k)
            drain_loads(k)
            drain_write(k)
            transpose(k, VCH)
            pltpu.async_copy(outbufs[k],
                             out_hbm.at[pl.ds(out_rows(u), VCH // 4)],
                             sem_ws[k])

            @pl.when(p + 1 < NP)
            def _():
                fire_loads(unit(p + 1, k), k)

    drain_write(0)
    drain_write(1)

    f = jnp.minimum(wid, F - 1)

    @pl.loop(0, 4)
    def _eld(dblk):
        pltpu.async_copy(
            tabt_hbm.at[f, pl.ds(dblk * 8, 8), pl.ds(VMAIN, VEPI)],
            staged0.at[dblk, :, pl.ds(0, VEPI)], sem_l0)

    @pl.loop(0, 4)
    def _edr(dblk):
        pltpu.make_async_copy(
            tabt_hbm.at[0, pl.ds(0, 8), pl.ds(VMAIN, VEPI)],
            staged0.at[0, :, pl.ds(0, VEPI)], sem_l0).wait()

    @plsc.parallel_loop(0, VEPI, unroll=8)
    def _ev(v):
        vb = jnp.zeros((LANES,), jnp.int32) + v
        row = v >> 2
        col = (v & 3) * D
        outbuf0[row, pl.ds(col, LANES)] = plsc.load_gather(
            staged0, [blk_lo, dsub, vb])
        outbuf0[row, pl.ds(col + LANES, LANES)] = plsc.load_gather(
            staged0, [blk_hi, dsub, vb])

    pltpu.sync_copy(outbuf0.at[pl.ds(0, VEPI // 4)],
                    out_hbm.at[pl.ds(f * MR + VMAIN // 4, VEPI // 4)])

    @pl.loop(wid, F, step=NW)
    def _tail(ff):
        pltpu.sync_copy(tail_hbm.at[ff],
                        out_hbm.at[pl.ds(ff * MR + VTAIL // 4, 8)])


def _gather_body(tab_hbm, x_hbm, out_hbm, x_v, idx_v, acc_v, sem):
    wid = lax.axis_index("s") * NUM_CORES + lax.axis_index("c")
    pltpu.sync_copy(x_hbm.at[pl.ds(wid * XPT, XPT)], x_v)

    lane_f = lax.iota(jnp.int32, 16) * F

    def _build_row(j):
        i = lax.div(j, NCH)
        c = lax.rem(j, NCH)
        base = c * CH * F + i
        off = i * V

        @pl.loop(0, CH // LANES)
        def _seg(r):
            xi = lane_f + (base + r * LANES * F)
            vals = plsc.load_gather(x_v, [xi])
            idx_v[j, pl.ds(r * LANES, LANES)] = vals + off

    @pl.loop(0, NCH)
    def _build0(j):
        _build_row(j)

    @pl.loop(0, NCH)
    def _init(c):
        pltpu.async_copy(tab_hbm.at[idx_v.at[c]],
                         acc_v.at[pl.ds(c * CH, CH)], sem)

    @pl.loop(NCH, F * NCH)
    def _build(j):
        _build_row(j)

    @pl.loop(0, NCH)
    def _init_drain(c):
        pltpu.make_async_copy(tab_hbm.at[idx_v.at[0]],
                              acc_v.at[pl.ds(0, CH)], sem).wait()

    @pl.loop(NCH, F * NCH)
    def _fire(j):
        c = lax.rem(j, NCH)
        pltpu.async_copy(tab_hbm.at[idx_v.at[j]],
                         acc_v.at[pl.ds(c * CH, CH)], sem, add=True)

    @pl.loop(NCH, F * NCH)
    def _drain(j):
        pltpu.make_async_copy(tab_hbm.at[idx_v.at[0]],
                              acc_v.at[pl.ds(0, CH)], sem).wait()

    pltpu.sync_copy(acc_v, out_hbm.at[pl.ds(wid * SPT, SPT)])


_SC_MESH = dict(core_axis_name="c", subcore_axis_name="s",
                num_cores=NUM_CORES, num_subcores=NUM_SUBCORES)


@jax.jit
def kernel(x, tables):
    tabt = jnp.swapaxes(tables, 1, 2)
    tail = tables[:, VTAIL:, :].reshape(F, 8, 4 * D)
    x_flat = x.reshape(B * F)

    conv = pl.kernel(
        _conv_body,
        out_type=jax.ShapeDtypeStruct((F * MR, 4 * D), jnp.float32),
        compiler_params=pltpu.CompilerParams(use_tc_tiling_on_sc=True,
                                             needs_layout_passes=False),
        mesh=plsc.VectorSubcoreMesh(**_SC_MESH),
        scratch_types=[
            pltpu.VMEM((4, 8, VCH + 1), jnp.float32),
            pltpu.VMEM((4, 8, VCH + 1), jnp.float32),
            pltpu.VMEM((VCH // 4, 4 * D), jnp.float32),
            pltpu.VMEM((VCH // 4, 4 * D), jnp.float32),
            pltpu.SemaphoreType.DMA,
            pltpu.SemaphoreType.DMA,
            pltpu.SemaphoreType.DMA,
            pltpu.SemaphoreType.DMA,
        ],
    )
    tab_l = conv(tabt, tail).reshape(F * V, D)

    gat = pl.kernel(
        _gather_body,
        out_type=jax.ShapeDtypeStruct((B, D), jnp.float32),
        compiler_params=pltpu.CompilerParams(use_tc_tiling_on_sc=False,
                                             needs_layout_passes=False),
        mesh=plsc.VectorSubcoreMesh(**_SC_MESH),
        scratch_types=[
            pltpu.VMEM((XPT,), jnp.int32),
            pltpu.VMEM((F * NCH, CH), jnp.int32),
            pltpu.VMEM((SPT, D), jnp.float32),
            pltpu.SemaphoreType.DMA,
        ],
    )
    return gat(tab_l, x_flat)


# speedup vs baseline: 1.5804x; 1.0004x over previous
import jax
import jax.numpy as jnp
from jax import lax
from jax.experimental import pallas as pl
from jax.experimental.pallas import tpu as pltpu
from jax.experimental.pallas import tpu_sc as plsc

NUM_CORES = 2
NUM_SUBCORES = 16
LANES = 16
NW = NUM_CORES * NUM_SUBCORES

F = 26
V = 100000
D = 32
B = 16384

MR = V // 4
VCH = 896
NCHV = 111
VEPI = 512
VMAIN = VCH * NCHV
VTAIL = VMAIN + VEPI

NU = F * NCHV
NP = (NU + 2 * NW - 1) // (2 * NW)

SPT = B // NW
CH = 128
NCH = SPT // CH
XPT = SPT * F


def _conv_body(tabt_hbm, tail_hbm, out_hbm,
               staged0, staged1, outbuf0, outbuf1,
               sem_l0, sem_l1, sem_w0, sem_w1):
    wid = lax.axis_index("s") * NUM_CORES + lax.axis_index("c")
    lane = lax.iota(jnp.int32, 16)
    stageds = (staged0, staged1)
    sem_ls = (sem_l0, sem_l1)
    outbufs = (outbuf0, outbuf1)
    sem_ws = (sem_w0, sem_w1)

    def unit(p, parity):
        return jnp.minimum(wid + (2 * p + parity) * NW, NU - 1)

    def fire_loads(u, k):
        f = lax.div(u, NCHV)
        v0 = lax.rem(u, NCHV) * VCH

        @pl.loop(0, 4)
        def _ld(dblk):
            pltpu.async_copy(
                tabt_hbm.at[f, pl.ds(dblk * 8, 8), pl.ds(v0, VCH)],
                stageds[k].at[dblk, :, pl.ds(0, VCH)], sem_ls[k])

    def drain_loads(k):
        @pl.loop(0, 4)
        def _dr(dblk):
            pltpu.make_async_copy(
                tabt_hbm.at[0, pl.ds(0, 8), pl.ds(0, VCH)],
                stageds[k].at[0, :, pl.ds(0, VCH)], sem_ls[k]).wait()

    def out_rows(u):
        f = lax.div(u, NCHV)
        v0 = lax.rem(u, NCHV) * VCH
        return f * MR + lax.div(v0, 4)

    blk_lo = lane >> 3
    blk_hi = blk_lo + 2
    dsub = lane & 7

    def transpose(k, nv):
        @plsc.parallel_loop(0, nv, unroll=16)
        def _v(v):
            vb = jnp.zeros((LANES,), jnp.int32) + v
            row = v >> 2
            col = (v & 3) * D
            outbufs[k][row, pl.ds(col, LANES)] = plsc.load_gather(
                stageds[k], [blk_lo, dsub, vb])
            outbufs[k][row, pl.ds(col + LANES, LANES)] = plsc.load_gather(
                stageds[k], [blk_hi, dsub, vb])

    def drain_write(k):
        pltpu.make_async_copy(out_hbm.at[pl.ds(0, VCH // 4)],
                              outbufs[k], sem_ws[k]).wait()

    fire_loads(unit(0, 0), 0)
    fire_loads(unit(0, 1), 1)
    pltpu.async_copy(out_hbm.at[pl.ds(0, VCH // 4)], outbuf0, sem_w0)
    pltpu.async_copy(out_hbm.at[pl.ds(0, VCH // 4)], outbuf1, sem_w1)

    @pl.loop(0, NP)
    def _pipe(p):
        for k in (0, 1):
            u = unit(p, k)
            drain_loads(k)
            drain_write(k)
            transpose(k, VCH)
            pltpu.async_copy(outbufs[k],
                             out_hbm.at[pl.ds(out_rows(u), VCH // 4)],
                             sem_ws[k])

            @pl.when(p + 1 < NP)
            def _():
                fire_loads(unit(p + 1, k), k)

    drain_write(0)
    drain_write(1)

    f = jnp.minimum(wid, F - 1)

    @pl.loop(0, 4)
    def _eld(dblk):
        pltpu.async_copy(
            tabt_hbm.at[f, pl.ds(dblk * 8, 8), pl.ds(VMAIN, VEPI)],
            staged0.at[dblk, :, pl.ds(0, VEPI)], sem_l0)

    @pl.loop(0, 4)
    def _edr(dblk):
        pltpu.make_async_copy(
            tabt_hbm.at[0, pl.ds(0, 8), pl.ds(VMAIN, VEPI)],
            staged0.at[0, :, pl.ds(0, VEPI)], sem_l0).wait()

    @plsc.parallel_loop(0, VEPI, unroll=16)
    def _ev(v):
        vb = jnp.zeros((LANES,), jnp.int32) + v
        row = v >> 2
        col = (v & 3) * D
        outbuf0[row, pl.ds(col, LANES)] = plsc.load_gather(
            staged0, [blk_lo, dsub, vb])
        outbuf0[row, pl.ds(col + LANES, LANES)] = plsc.load_gather(
            staged0, [blk_hi, dsub, vb])

    pltpu.sync_copy(outbuf0.at[pl.ds(0, VEPI // 4)],
                    out_hbm.at[pl.ds(f * MR + VMAIN // 4, VEPI // 4)])

    @pl.loop(wid, F, step=NW)
    def _tail(ff):
        pltpu.sync_copy(tail_hbm.at[ff],
                        out_hbm.at[pl.ds(ff * MR + VTAIL // 4, 8)])


def _gather_body(tab_hbm, x_hbm, out_hbm, x_v, idx_v, acc_v, sem):
    wid = lax.axis_index("s") * NUM_CORES + lax.axis_index("c")
    pltpu.sync_copy(x_hbm.at[pl.ds(wid * XPT, XPT)], x_v)

    lane_f = lax.iota(jnp.int32, 16) * F

    def _build_row(j):
        i = lax.div(j, NCH)
        c = lax.rem(j, NCH)
        base = c * CH * F + i
        off = i * V

        @pl.loop(0, CH // LANES)
        def _seg(r):
            xi = lane_f + (base + r * LANES * F)
            vals = plsc.load_gather(x_v, [xi])
            idx_v[j, pl.ds(r * LANES, LANES)] = vals + off

    @pl.loop(0, NCH)
    def _build0(j):
        _build_row(j)

    @pl.loop(0, NCH)
    def _init(c):
        pltpu.async_copy(tab_hbm.at[idx_v.at[c]],
                         acc_v.at[pl.ds(c * CH, CH)], sem)

    @pl.loop(NCH, F * NCH)
    def _build(j):
        _build_row(j)

    @pl.loop(0, NCH)
    def _init_drain(c):
        pltpu.make_async_copy(tab_hbm.at[idx_v.at[0]],
                              acc_v.at[pl.ds(0, CH)], sem).wait()

    @pl.loop(NCH, F * NCH)
    def _fire(j):
        c = lax.rem(j, NCH)
        pltpu.async_copy(tab_hbm.at[idx_v.at[j]],
                         acc_v.at[pl.ds(c * CH, CH)], sem, add=True)

    @pl.loop(NCH, F * NCH)
    def _drain(j):
        pltpu.make_async_copy(tab_hbm.at[idx_v.at[0]],
                              acc_v.at[pl.ds(0, CH)], sem).wait()

    pltpu.sync_copy(acc_v, out_hbm.at[pl.ds(wid * SPT, SPT)])


_SC_MESH = dict(core_axis_name="c", subcore_axis_name="s",
                num_cores=NUM_CORES, num_subcores=NUM_SUBCORES)


@jax.jit
def kernel(x, tables):
    tabt = jnp.swapaxes(tables, 1, 2)
    tail = tables[:, VTAIL:, :].reshape(F, 8, 4 * D)
    x_flat = x.reshape(B * F)

    conv = pl.kernel(
        _conv_body,
        out_type=jax.ShapeDtypeStruct((F * MR, 4 * D), jnp.float32),
        compiler_params=pltpu.CompilerParams(use_tc_tiling_on_sc=True,
                                             needs_layout_passes=False),
        mesh=plsc.VectorSubcoreMesh(**_SC_MESH),
        scratch_types=[
            pltpu.VMEM((4, 8, VCH + 1), jnp.float32),
            pltpu.VMEM((4, 8, VCH + 1), jnp.float32),
            pltpu.VMEM((VCH // 4, 4 * D), jnp.float32),
            pltpu.VMEM((VCH // 4, 4 * D), jnp.float32),
            pltpu.SemaphoreType.DMA,
            pltpu.SemaphoreType.DMA,
            pltpu.SemaphoreType.DMA,
            pltpu.SemaphoreType.DMA,
        ],
    )
    tab_l = conv(tabt, tail).reshape(F * V, D)

    gat = pl.kernel(
        _gather_body,
        out_type=jax.ShapeDtypeStruct((B, D), jnp.float32),
        compiler_params=pltpu.CompilerParams(use_tc_tiling_on_sc=False,
                                             needs_layout_passes=False),
        mesh=plsc.VectorSubcoreMesh(**_SC_MESH),
        scratch_types=[
            pltpu.VMEM((XPT,), jnp.int32),
            pltpu.VMEM((F * NCH, CH), jnp.int32),
            pltpu.VMEM((SPT, D), jnp.float32),
            pltpu.SemaphoreType.DMA,
        ],
    )
    return gat(tab_l, x_flat)
